# baseline (speedup 1.0000x reference)
import functools

import jax
import jax.numpy as jnp
from jax import lax
from jax.experimental import pallas as pl
from jax.experimental.pallas import tpu as pltpu
from jax.experimental.pallas import tpu_sc as plsc

_NC = 2
_NS = 16
_NW = _NC * _NS
_L = 16
_UNROLL = 4

_LN2 = 0.6931471805599453
_C5 = 0.030449004538686555
_C4 = -0.13158182508879562
_C3 = 0.28527268109059173
_C2 = -0.49023072342341184
_C1 = 0.9992354838332752
_C0 = 9.975032552169407e-06


def _softlog(x):
    bits = lax.bitcast_convert_type(x, jnp.int32)
    e = (bits >> 23) - 127
    mant = lax.bitcast_convert_type((bits & 0x007FFFFF) | 0x3F800000,
                                    jnp.float32)
    t = mant - 1.0
    p = ((((_C5 * t + _C4) * t + _C3) * t + _C2) * t + _C1) * t + _C0
    return e.astype(jnp.float32) * _LN2 + p


def _sc_partials_body(yp_hbm, yt_hbm, mf_hbm, out_hbm,
                      yp_v, yt_v, mf_v, acc_v, sem):
    chunk = yp_v.shape[0]
    ngrp = chunk // (_UNROLL * _L)
    wid = lax.axis_index("s") * _NC + lax.axis_index("c")
    base = wid * chunk
    cp0 = pltpu.async_copy(yp_hbm.at[pl.ds(base, chunk)], yp_v, sem)
    cp1 = pltpu.async_copy(yt_hbm.at[pl.ds(base, chunk)], yt_v, sem)
    cp2 = pltpu.async_copy(mf_hbm.at[pl.ds(base, chunk)], mf_v, sem)
    cp0.wait()
    cp1.wait()
    cp2.wait()

    def body(g, carry):
        n, sp, st, spp, stt, spt, slt = carry
        for j in range(_UNROLL):
            off = (g * _UNROLL + j) * _L
            p = yp_v[pl.ds(off, _L)]
            t = yt_v[pl.ds(off, _L)]
            m = mf_v[pl.ds(off, _L)]
            pm = p * m
            tm = t * m
            lg = _softlog(p + 1e-8)
            n = n + m
            sp = sp + pm
            st = st + tm
            spp = spp + p * pm
            stt = stt + t * tm
            spt = spt + p * tm
            slt = slt + lg * tm
        return (n, sp, st, spp, stt, spt, slt)

    zf = jnp.zeros((_L,), jnp.float32)
    accs = lax.fori_loop(0, ngrp, body, (zf,) * 7)
    for j in range(7):
        acc_v[pl.ds(j * _L, _L)] = accs[j]
    pltpu.sync_copy(acc_v, out_hbm.at[wid])


def _sc_partials(yp, yt, mf):
    chunk = yp.shape[0] // _NW
    fn = functools.partial(
        pl.kernel,
        mesh=plsc.VectorSubcoreMesh(core_axis_name="c", subcore_axis_name="s"),
        out_type=jax.ShapeDtypeStruct((_NW, 7 * _L), jnp.float32),
        scratch_types=[
            pltpu.VMEM((chunk,), jnp.float32),
            pltpu.VMEM((chunk,), jnp.float32),
            pltpu.VMEM((chunk,), jnp.float32),
            pltpu.VMEM((7 * _L,), jnp.float32),
            pltpu.SemaphoreType.DMA,
        ],
    )(_sc_partials_body)
    return fn(yp, yt, mf)


def _tc_partials_body(yp_ref, yt_ref, mk_ref, out_ref):
    p = yp_ref[...]
    t = yt_ref[...]
    m = mk_ref[...].astype(jnp.float32)
    pm = p * m
    tm = t * m
    out_ref[0, 0] = jnp.sum(m)
    out_ref[0, 1] = jnp.sum(pm)
    out_ref[0, 2] = jnp.sum(tm)
    out_ref[0, 3] = jnp.sum(p * pm)
    out_ref[0, 4] = jnp.sum(t * tm)
    out_ref[0, 5] = jnp.sum(p * tm)
    out_ref[0, 6] = jnp.sum(jnp.log(p + 1e-8) * tm)
    out_ref[0, 7] = 0.0


def _tc_partials(yp2, yt2, mk2):
    return pl.pallas_call(
        _tc_partials_body,
        out_shape=jax.ShapeDtypeStruct((1, 8), jnp.float32),
        out_specs=pl.BlockSpec(memory_space=pltpu.SMEM),
    )(yp2, yt2, mk2)


def _tc_finalize_body(ts_ref, parts_ref, tcp_ref, out_ref):
    parts = parts_ref[...]
    s = [jnp.sum(parts[:, j * _L:(j + 1) * _L]) + tcp_ref[0, j]
         for j in range(7)]
    n, sp, st, spp, stt, spt, slt = s
    pois = sp - slt
    eps = 1e-6
    mx = sp / n
    my = st / n
    num = spt - n * mx * my
    nx = jnp.sqrt(jnp.maximum(spp - n * mx * mx, 0.0))
    ny = jnp.sqrt(jnp.maximum(stt - n * my * my, 0.0))
    cos = num / (jnp.maximum(nx, eps) * jnp.maximum(ny, eps))
    w = jnp.maximum(0.0, 1.0 - ts_ref[0, 0] / 10.0)
    out_ref[...] = jnp.full((1, 1), (1.0 - cos) + w * (pois / n), jnp.float32)


def _tc_finalize(ts, sc_parts, tc_parts):
    return pl.pallas_call(
        _tc_finalize_body,
        out_shape=jax.ShapeDtypeStruct((1, 1), jnp.float32),
        in_specs=[
            pl.BlockSpec(memory_space=pltpu.SMEM),
            pl.BlockSpec(memory_space=pltpu.VMEM),
            pl.BlockSpec(memory_space=pltpu.SMEM),
        ],
        out_specs=pl.BlockSpec(memory_space=pltpu.VMEM),
    )(ts, sc_parts, tc_parts)


def kernel(y_pred, y_true, mask, timestamp):
    nhalf = y_pred.size // 2
    yp = y_pred.reshape(-1)[:nhalf]
    yt = y_true.reshape(-1)[:nhalf]
    mf = mask.reshape(-1)[:nhalf].astype(jnp.float32)
    half = y_pred.shape[0] // 2
    sc_parts = _sc_partials(yp, yt, mf)
    tc_parts = _tc_partials(y_pred[half:], y_true[half:], mask[half:])
    ts = jnp.asarray(timestamp, jnp.float32).reshape(1, 1)
    return _tc_finalize(ts, sc_parts, tc_parts).reshape(())


# speedup vs baseline: 1.0514x; 1.0514x over previous
import functools

import jax
import jax.numpy as jnp
from jax import lax
from jax.experimental import pallas as pl
from jax.experimental.pallas import tpu as pltpu
from jax.experimental.pallas import tpu_sc as plsc

_NC = 2
_NS = 16
_NW = _NC * _NS
_L = 16
_UNROLL = 4

_LN2 = 0.6931471805599453
_C5 = 0.030449004538686555
_C4 = -0.13158182508879562
_C3 = 0.28527268109059173
_C2 = -0.49023072342341184
_C1 = 0.9992354838332752
_C0 = 9.975032552169407e-06


def _softlog(x):
    bits = lax.bitcast_convert_type(x, jnp.int32)
    e = (bits >> 23) - 127
    mant = lax.bitcast_convert_type((bits & 0x007FFFFF) | 0x3F800000,
                                    jnp.float32)
    t = mant - 1.0
    p = ((((_C5 * t + _C4) * t + _C3) * t + _C2) * t + _C1) * t + _C0
    return e.astype(jnp.float32) * _LN2 + p


def _sc_partials_body(yp_hbm, yt_hbm, mf_hbm, out_hbm,
                      yp_v, yt_v, mf_v, acc_v, sem):
    chunk = yp_v.shape[0]
    ngrp = chunk // (_UNROLL * _L)
    wid = lax.axis_index("s") * _NC + lax.axis_index("c")
    base = wid * chunk
    cp0 = pltpu.async_copy(yp_hbm.at[pl.ds(base, chunk)], yp_v, sem)
    cp1 = pltpu.async_copy(yt_hbm.at[pl.ds(base, chunk)], yt_v, sem)
    cp2 = pltpu.async_copy(mf_hbm.at[pl.ds(base, chunk)], mf_v, sem)
    cp0.wait()
    cp1.wait()
    cp2.wait()

    def body(g, carry):
        n, sp, st, spp, stt, spt, slt = carry
        for j in range(_UNROLL):
            off = (g * _UNROLL + j) * _L
            p = yp_v[pl.ds(off, _L)]
            t = yt_v[pl.ds(off, _L)]
            m = mf_v[pl.ds(off, _L)]
            pm = p * m
            tm = t * m
            lg = _softlog(p + 1e-8)
            n = n + m
            sp = sp + pm
            st = st + tm
            spp = spp + p * pm
            stt = stt + t * tm
            spt = spt + p * tm
            slt = slt + lg * tm
        return (n, sp, st, spp, stt, spt, slt)

    zf = jnp.zeros((_L,), jnp.float32)
    accs = lax.fori_loop(0, ngrp, body, (zf,) * 7)
    for j in range(7):
        acc_v[pl.ds(j * _L, _L)] = accs[j]
    pltpu.sync_copy(acc_v, out_hbm.at[wid])


def _sc_partials(yp, yt, mf, nelems):
    chunk = nelems // _NW
    fn = functools.partial(
        pl.kernel,
        mesh=plsc.VectorSubcoreMesh(core_axis_name="c", subcore_axis_name="s"),
        out_type=jax.ShapeDtypeStruct((_NW, 7 * _L), jnp.float32),
        scratch_types=[
            pltpu.VMEM((chunk,), jnp.float32),
            pltpu.VMEM((chunk,), jnp.float32),
            pltpu.VMEM((chunk,), jnp.float32),
            pltpu.VMEM((7 * _L,), jnp.float32),
            pltpu.SemaphoreType.DMA,
        ],
    )(_sc_partials_body)
    return fn(yp, yt, mf)


def _tc_partials_body(yp_ref, yt_ref, mk_ref, out_ref):
    p = yp_ref[...]
    t = yt_ref[...]
    m = mk_ref[...].astype(jnp.float32)
    pm = p * m
    tm = t * m
    out_ref[0, 0] = jnp.sum(m)
    out_ref[0, 1] = jnp.sum(pm)
    out_ref[0, 2] = jnp.sum(tm)
    out_ref[0, 3] = jnp.sum(p * pm)
    out_ref[0, 4] = jnp.sum(t * tm)
    out_ref[0, 5] = jnp.sum(p * tm)
    out_ref[0, 6] = jnp.sum(jnp.log(p + 1e-8) * tm)
    out_ref[0, 7] = 0.0


def _tc_partials(y_pred, y_true, mask):
    half = y_pred.shape[0] // 2
    spec = pl.BlockSpec((half, y_pred.shape[1]), lambda i: (1, 0))
    return pl.pallas_call(
        _tc_partials_body,
        grid=(1,),
        out_shape=jax.ShapeDtypeStruct((1, 8), jnp.float32),
        in_specs=[spec, spec, spec],
        out_specs=pl.BlockSpec((1, 8), lambda i: (0, 0),
                               memory_space=pltpu.SMEM),
    )(y_pred, y_true, mask)


def _tc_finalize_body(ts_ref, parts_ref, tcp_ref, out_ref):
    parts = parts_ref[...]
    s = [jnp.sum(parts[:, j * _L:(j + 1) * _L]) + tcp_ref[0, j]
         for j in range(7)]
    n, sp, st, spp, stt, spt, slt = s
    pois = sp - slt
    eps = 1e-6
    mx = sp / n
    my = st / n
    num = spt - n * mx * my
    nx = jnp.sqrt(jnp.maximum(spp - n * mx * mx, 0.0))
    ny = jnp.sqrt(jnp.maximum(stt - n * my * my, 0.0))
    cos = num / (jnp.maximum(nx, eps) * jnp.maximum(ny, eps))
    w = jnp.maximum(0.0, 1.0 - ts_ref[0, 0] / 10.0)
    out_ref[...] = jnp.full((1, 1), (1.0 - cos) + w * (pois / n), jnp.float32)


def _tc_finalize(ts, sc_parts, tc_parts):
    return pl.pallas_call(
        _tc_finalize_body,
        out_shape=jax.ShapeDtypeStruct((1, 1), jnp.float32),
        in_specs=[
            pl.BlockSpec(memory_space=pltpu.SMEM),
            pl.BlockSpec(memory_space=pltpu.VMEM),
            pl.BlockSpec(memory_space=pltpu.SMEM),
        ],
        out_specs=pl.BlockSpec(memory_space=pltpu.VMEM),
    )(ts, sc_parts, tc_parts)


def kernel(y_pred, y_true, mask, timestamp):
    yp = y_pred.reshape(-1)
    yt = y_true.reshape(-1)
    mf = mask.reshape(-1).astype(jnp.float32)
    sc_parts = _sc_partials(yp, yt, mf, y_pred.size // 2)
    tc_parts = _tc_partials(y_pred, y_true, mask)
    ts = jnp.asarray(timestamp, jnp.float32).reshape(1, 1)
    return _tc_finalize(ts, sc_parts, tc_parts).reshape(())
